# baseline (device time: 115637 ns/iter reference)
import jax
import jax.numpy as jnp
from jax import lax
from jax.experimental import pallas as pl
from jax.experimental.pallas import tpu as pltpu

EPS = 1e-5
N_GLOBAL = 4096.0
BM = 512


def _partial_body(x_ref, p_ref):
    x = x_ref[...]
    p_ref[...] = jnp.sum(x * x, axis=1, keepdims=True)


def _exchange_body(p_ref, o_ref, comm_ref, send_sem, recv_sem):
    my_x = lax.axis_index("x")
    my_y = lax.axis_index("y")
    peer = (my_x, 1 - my_y)

    barrier_sem = pltpu.get_barrier_semaphore()
    pl.semaphore_signal(
        barrier_sem, inc=1, device_id=peer,
        device_id_type=pl.DeviceIdType.MESH,
    )
    pl.semaphore_wait(barrier_sem, 1)

    rdma = pltpu.make_async_remote_copy(
        src_ref=p_ref,
        dst_ref=comm_ref,
        send_sem=send_sem,
        recv_sem=recv_sem,
        device_id=peer,
        device_id_type=pl.DeviceIdType.MESH,
    )
    rdma.start()
    rdma.wait()

    total = p_ref[...] + comm_ref[...]
    o_ref[...] = lax.rsqrt(total * (1.0 / N_GLOBAL) + EPS)


def _apply_body(x_ref, g_ref, r_ref, o_ref):
    o_ref[...] = x_ref[...] * r_ref[...] * g_ref[...]


def kernel(x, gamma):
    m, n = x.shape
    nb = m // BM

    partial = pl.pallas_call(
        _partial_body,
        grid=(nb,),
        in_specs=[pl.BlockSpec((BM, n), lambda i: (i, 0))],
        out_specs=pl.BlockSpec((BM, 1), lambda i: (i, 0)),
        out_shape=jax.ShapeDtypeStruct((m, 1), jnp.float32),
    )(x)

    inv_rms = pl.pallas_call(
        _exchange_body,
        in_specs=[pl.BlockSpec(memory_space=pltpu.VMEM)],
        out_specs=pl.BlockSpec(memory_space=pltpu.VMEM),
        out_shape=jax.ShapeDtypeStruct((m, 1), jnp.float32),
        scratch_shapes=[
            pltpu.VMEM((m, 1), jnp.float32),
            pltpu.SemaphoreType.DMA,
            pltpu.SemaphoreType.DMA,
        ],
        compiler_params=pltpu.CompilerParams(collective_id=0),
    )(partial)

    out = pl.pallas_call(
        _apply_body,
        grid=(nb,),
        in_specs=[
            pl.BlockSpec((BM, n), lambda i: (i, 0)),
            pl.BlockSpec((n,), lambda i: (0,)),
            pl.BlockSpec((BM, 1), lambda i: (i, 0)),
        ],
        out_specs=pl.BlockSpec((BM, n), lambda i: (i, 0)),
        out_shape=jax.ShapeDtypeStruct((m, n), jnp.float32),
    )(x, gamma, inv_rms)

    return out
